# baseline (speedup 1.0000x reference)
import numpy as np
import jax
import jax.numpy as jnp
from jax.experimental import pallas as pl
from jax.experimental.pallas import tpu as pltpu

_ROI = 9
_P = _ROI * _ROI
_GROUPS = 32
F32 = jnp.float32
BF16 = jnp.bfloat16


def _shift_mats(side):
    n = side * side
    t = np.arange(n)
    h, w = t // side, t % side
    mats = np.zeros((9, n, n), np.float32)
    for ky in range(3):
        for kx in range(3):
            hs, ws = h + ky - 1, w + kx - 1
            valid = (hs >= 0) & (hs < side) & (ws >= 0) & (ws < side)
            src = np.clip(hs, 0, side - 1) * side + np.clip(ws, 0, side - 1)
            mats[ky * 3 + kx][t[valid], src[valid]] = 1.0
    return mats


def _group_mat(n_ch):
    G = np.zeros((n_ch, _GROUPS), np.float32)
    G[np.arange(n_ch), np.arange(n_ch) // (n_ch // _GROUPS)] = 1.0
    return G


_SH16 = _shift_mats(16)
_SH9 = _shift_mats(9)
_GL = _group_mat(320)
_GC = _group_mat(1024)


def _iota(shape, dim):
    return jax.lax.broadcasted_iota(jnp.int32, shape, dim).astype(F32)


_HI = jax.lax.Precision.HIGHEST


def _gn_stats(x, G_ref, GT_ref, n, eps):
    s = jnp.dot(jnp.sum(x, axis=0, keepdims=True), G_ref[...],
                precision=_HI, preferred_element_type=F32)
    sq = jnp.dot(jnp.sum(x * x, axis=0, keepdims=True), G_ref[...],
                 precision=_HI, preferred_element_type=F32)
    mu = s / n
    var = sq / n - mu * mu
    inv = jax.lax.rsqrt(var + eps)
    mu_c = jnp.dot(mu, GT_ref[...], precision=_HI, preferred_element_type=F32)
    inv_c = jnp.dot(inv, GT_ref[...], precision=_HI,
                    preferred_element_type=F32)
    return mu_c, inv_c


def _refine_kernel(gc_ref, ctx_ref, ind_ref, bbox_ref, cond_ref,
                   Wqm_ref, Wqi_ref, bq_ref, Wk_ref, bk_ref,
                   Wv_ref, bv_ref, Wsg_ref, bsg_ref, Wsb_ref, bsb_ref,
                   lng_ref, lnb_ref, cng_ref, cnb_ref,
                   GL_ref, GLT_ref, GC_ref, GCT_ref, SH16_ref, SH9_ref,
                   xl_ref, attn_ref):
    T, D = ctx_ref.shape[1], ctx_ref.shape[2]
    C = Wqm_ref.shape[0]

    gc = gc_ref[0]
    ctx = ctx_ref[0]
    ind = ind_ref[0]
    bbox = bbox_ref[0]
    cond = cond_ref[0]

    ii = _iota((_ROI, _ROI, 64), 0)
    jj = _iota((_ROI, _ROI, 64), 1)
    ss = _iota((_ROI, _ROI, 64), 2)
    yy = jnp.floor(ss * 0.125)
    xx = ss - yy * 8.0
    x1 = bbox[:, 0:1]
    y1 = bbox[:, 1:2]
    rw = jnp.maximum(bbox[:, 2:3] - x1, 1.0)
    rh = jnp.maximum(bbox[:, 3:4] - y1, 1.0)
    yc = jnp.clip(y1 + (ii + 0.5) / 9.0 * rh, 0.0, 63.0)
    xc = jnp.clip(x1 + (jj + 0.5) / 9.0 * rw, 0.0, 63.0)
    y0 = jnp.floor(yc)
    ly = yc - y0
    y1i = jnp.minimum(y0 + 1.0, 63.0)
    x0 = jnp.floor(xc)
    lx = xc - x0
    x1i = jnp.minimum(x0 + 1.0, 63.0)
    wy = jnp.where(yy == y0, 1.0 - ly, 0.0) + jnp.where(yy == y1i, ly, 0.0)
    wx = jnp.where(xx == x0, 1.0 - lx, 0.0) + jnp.where(xx == x1i, lx, 0.0)
    S_T = (wy * wx).reshape(_P, 64)
    roi = jax.lax.dot_general(S_T, gc, (((1,), (1,)), ((), ())),
                              precision=_HI,
                              preferred_element_type=F32)

    mu, inv = _gn_stats(roi, GL_ref, GLT_ref, _P * (C // _GROUPS), 1e-6)
    xn = (roi - mu) * inv * lng_ref[...] + lnb_ref[...]
    q = (jnp.dot(xn, Wqm_ref[...], preferred_element_type=F32)
         + jnp.dot(ind, Wqi_ref[...], preferred_element_type=F32)
         + bq_ref[...])

    mean_ctx = jnp.sum(ctx, axis=0, keepdims=True) / T
    ce = cond * ctx + (1.0 - cond) * mean_ctx

    mu2, inv2 = _gn_stats(ce, GC_ref, GCT_ref, T * (D // _GROUPS), 1e-6)
    ckn = (ce - mu2) * inv2 * cng_ref[...] + cnb_ref[...]
    k = jnp.dot(ckn, Wk_ref[...], preferred_element_type=F32) + bk_ref[...]

    ce_bf = ce.astype(BF16)
    v = bv_ref[...]
    for tap in range(9):
        y = jnp.dot(ce_bf, Wv_ref[tap],
                    preferred_element_type=F32).astype(BF16)
        v = v + jnp.dot(SH16_ref[tap], y, preferred_element_type=F32)

    logits = jax.lax.dot_general(q, k, (((1,), (1,)), ((), ())),
                                 preferred_element_type=F32)
    m = jnp.max(logits, axis=1, keepdims=True)
    e = jnp.exp(logits - m)
    attn = e / jnp.sum(e, axis=1, keepdims=True)
    attn_ref[0] = attn

    xa = jnp.dot(attn, v, preferred_element_type=F32)
    ac = jnp.dot(attn, ce, preferred_element_type=F32)

    mu3, inv3 = _gn_stats(xa, GL_ref, GLT_ref, _P * (C // _GROUPS), 1e-5)
    xn2 = (xa - mu3) * inv3
    ac_bf = ac.astype(BF16)
    sg = bsg_ref[...]
    sb = bsb_ref[...]
    for tap in range(9):
        yg = jnp.dot(ac_bf, Wsg_ref[tap],
                     preferred_element_type=F32).astype(BF16)
        yb = jnp.dot(ac_bf, Wsb_ref[tap],
                     preferred_element_type=F32).astype(BF16)
        sg = sg + jnp.dot(SH9_ref[tap], yg, preferred_element_type=F32)
        sb = sb + jnp.dot(SH9_ref[tap], yb, preferred_element_type=F32)
    xl_ref[0] = jnp.transpose(xn2 * sg + sb)


def _paste_kernel(gx_ref, xl_ref, bbox_ref, out_ref):
    gx = gx_ref[0]
    xlT = xl_ref[0]
    bbox = bbox_ref[0]
    x1b = jnp.floor(bbox[:, 0:1] * 64.0)
    y1b = jnp.floor(bbox[:, 1:2] * 64.0)
    x2b = jnp.maximum(jnp.floor(bbox[:, 2:3] * 64.0), x1b + 1.0)
    y2b = jnp.maximum(jnp.floor(bbox[:, 3:4] * 64.0), y1b + 1.0)
    oh = y2b - y1b
    ow = x2b - x1b
    i3 = _iota((_ROI, _ROI, 4096), 0)
    j3 = _iota((_ROI, _ROI, 4096), 1)
    q3 = _iota((_ROI, _ROI, 4096), 2)
    y = jnp.floor(q3 * 0.015625)
    x = q3 - y * 64.0
    jy9 = (y - y1b) * 9.0
    jx9 = (x - x1b) * 9.0
    lo_y = i3 * oh
    lo_x = j3 * ow
    hit = ((jy9 >= lo_y) & (jy9 < lo_y + oh) & (y < y2b)
           & (jx9 >= lo_x) & (jx9 < lo_x + ow) & (x < x2b))
    Pm = jnp.where(hit, 1.0, 0.0).reshape(_P, 4096)
    out_ref[0] = gx + jnp.dot(xlT, Pm, preferred_element_type=F32)


def kernel(global_x, context, indicator, bbox, cond_flag,
           ln_g, ln_b, cn_g, cn_b, Wq, bq, Wk, bk, Wv, bv,
           Wsg, bsg, Wsb, bsb):
    B, C, H, W = global_x.shape
    T, D = context.shape[1], context.shape[2]
    HW = H * W

    gcorner = global_x[:, :, :8, :8].reshape(B, C, 64)
    gxf = global_x.reshape(B, C, HW)
    ind3 = indicator.reshape(B, 1, 2)
    bbox3 = bbox.reshape(B, 1, 4)
    cond3 = cond_flag.astype(F32).reshape(B, 1, 1)
    Wqm = Wq[:, :C, 0, 0].T
    Wqi = Wq[:, C:, 0, 0].T
    Wk2 = Wk[:, :, 0, 0].T
    Wv2 = Wv.transpose(2, 3, 1, 0).reshape(9, D, C).astype(BF16)
    Wsg2 = Wsg.transpose(2, 3, 1, 0).reshape(9, D, C).astype(BF16)
    Wsb2 = Wsb.transpose(2, 3, 1, 0).reshape(9, D, C).astype(BF16)
    row = lambda a: a.reshape(1, -1)

    consts = [jnp.asarray(_GL), jnp.asarray(_GL.T),
              jnp.asarray(_GC), jnp.asarray(_GC.T),
              jnp.asarray(_SH16, BF16), jnp.asarray(_SH9, BF16)]

    full = lambda s: pl.BlockSpec(s, lambda i, _n=None: (0,) * len(s))
    bat = lambda s: pl.BlockSpec((1,) + s, lambda i: (i,) + (0,) * len(s))

    cp = pltpu.CompilerParams(dimension_semantics=("parallel",),
                              vmem_limit_bytes=56 * 1024 * 1024)

    xlT, attn = pl.pallas_call(
        _refine_kernel,
        grid=(B,),
        in_specs=[bat((C, 64)), bat((T, D)), bat((1, 2)), bat((1, 4)),
                  bat((1, 1)),
                  full((C, C)), full((2, C)), full((1, C)),
                  full((D, C)), full((1, C)),
                  full((9, D, C)), full((1, C)),
                  full((9, D, C)), full((1, C)),
                  full((9, D, C)), full((1, C)),
                  full((1, C)), full((1, C)), full((1, D)), full((1, D)),
                  full((C, _GROUPS)), full((_GROUPS, C)),
                  full((D, _GROUPS)), full((_GROUPS, D)),
                  full((9, T, T)), full((9, _P, _P))],
        out_specs=[bat((C, _P)), bat((_P, T))],
        out_shape=[jax.ShapeDtypeStruct((B, C, _P), F32),
                   jax.ShapeDtypeStruct((B, _P, T), F32)],
        compiler_params=cp,
    )(gcorner, context, ind3, bbox3, cond3,
      Wqm, Wqi, row(bq), Wk2, row(bk),
      Wv2, row(bv), Wsg2, row(bsg), Wsb2, row(bsb),
      row(ln_g), row(ln_b), row(cn_g), row(cn_b), *consts)

    outf = pl.pallas_call(
        _paste_kernel,
        grid=(B,),
        in_specs=[bat((C, HW)), bat((C, _P)), bat((1, 4))],
        out_specs=bat((C, HW)),
        out_shape=jax.ShapeDtypeStruct((B, C, HW), F32),
        compiler_params=cp,
    )(gxf, xlT, bbox3)

    return outf.reshape(B, C, H, W), attn


# speedup vs baseline: 1.0918x; 1.0918x over previous
import numpy as np
import jax
import jax.numpy as jnp
from jax.experimental import pallas as pl
from jax.experimental.pallas import tpu as pltpu

_ROI = 9
_P = _ROI * _ROI
_GROUPS = 32
F32 = jnp.float32
BF16 = jnp.bfloat16


def _shift_mats(side):
    n = side * side
    t = np.arange(n)
    h, w = t // side, t % side
    mats = np.zeros((9, n, n), np.float32)
    for ky in range(3):
        for kx in range(3):
            hs, ws = h + ky - 1, w + kx - 1
            valid = (hs >= 0) & (hs < side) & (ws >= 0) & (ws < side)
            src = np.clip(hs, 0, side - 1) * side + np.clip(ws, 0, side - 1)
            mats[ky * 3 + kx][t[valid], src[valid]] = 1.0
    return mats


def _group_mat(n_ch):
    G = np.zeros((n_ch, _GROUPS), np.float32)
    G[np.arange(n_ch), np.arange(n_ch) // (n_ch // _GROUPS)] = 1.0
    return G


_SH16 = _shift_mats(16)
_SH9 = _shift_mats(9)
_GL = _group_mat(320)
_GC = _group_mat(1024)


def _iota(shape, dim):
    return jax.lax.broadcasted_iota(jnp.int32, shape, dim).astype(F32)


_HI = jax.lax.Precision.HIGHEST


def _gn_stats(x, G_ref, GT_ref, n, eps):
    s = jnp.dot(jnp.sum(x, axis=0, keepdims=True), G_ref[...],
                precision=_HI, preferred_element_type=F32)
    sq = jnp.dot(jnp.sum(x * x, axis=0, keepdims=True), G_ref[...],
                 precision=_HI, preferred_element_type=F32)
    mu = s / n
    var = sq / n - mu * mu
    inv = jax.lax.rsqrt(var + eps)
    mu_c = jnp.dot(mu, GT_ref[...], precision=_HI, preferred_element_type=F32)
    inv_c = jnp.dot(inv, GT_ref[...], precision=_HI,
                    preferred_element_type=F32)
    return mu_c, inv_c


def _refine_kernel(gc_ref, gx_ref, ctx_ref, ind_ref, bbox_ref, cond_ref,
                   Wqm_ref, Wqi_ref, bq_ref, Wk_ref, bk_ref,
                   Wv_ref, bv_ref, Wsg_ref, bsg_ref, Wsb_ref, bsb_ref,
                   lng_ref, lnb_ref, cng_ref, cnb_ref,
                   GL_ref, GLT_ref, GC_ref, GCT_ref, SH16_ref, SH9_ref,
                   out_ref, attn_ref):
    T, D = ctx_ref.shape[1], ctx_ref.shape[2]
    C = Wqm_ref.shape[0]

    gc = gc_ref[0]
    ctx = ctx_ref[0]
    ind = ind_ref[0]
    bbox = bbox_ref[0]
    cond = cond_ref[0]

    ii = _iota((_ROI, _ROI, 64), 0)
    jj = _iota((_ROI, _ROI, 64), 1)
    ss = _iota((_ROI, _ROI, 64), 2)
    yy = jnp.floor(ss * 0.125)
    xx = ss - yy * 8.0
    x1 = bbox[:, 0:1]
    y1 = bbox[:, 1:2]
    rw = jnp.maximum(bbox[:, 2:3] - x1, 1.0)
    rh = jnp.maximum(bbox[:, 3:4] - y1, 1.0)
    yc = jnp.clip(y1 + (ii + 0.5) / 9.0 * rh, 0.0, 63.0)
    xc = jnp.clip(x1 + (jj + 0.5) / 9.0 * rw, 0.0, 63.0)
    y0 = jnp.floor(yc)
    ly = yc - y0
    y1i = jnp.minimum(y0 + 1.0, 63.0)
    x0 = jnp.floor(xc)
    lx = xc - x0
    x1i = jnp.minimum(x0 + 1.0, 63.0)
    wy = jnp.where(yy == y0, 1.0 - ly, 0.0) + jnp.where(yy == y1i, ly, 0.0)
    wx = jnp.where(xx == x0, 1.0 - lx, 0.0) + jnp.where(xx == x1i, lx, 0.0)
    S_T = (wy * wx).reshape(_P, 64)
    roi = jax.lax.dot_general(S_T, gc, (((1,), (1,)), ((), ())),
                              precision=_HI,
                              preferred_element_type=F32)

    mu, inv = _gn_stats(roi, GL_ref, GLT_ref, _P * (C // _GROUPS), 1e-6)
    xn = (roi - mu) * inv * lng_ref[...] + lnb_ref[...]
    q = (jnp.dot(xn, Wqm_ref[...], preferred_element_type=F32)
         + jnp.dot(ind, Wqi_ref[...], preferred_element_type=F32)
         + bq_ref[...])

    mean_ctx = jnp.sum(ctx, axis=0, keepdims=True) / T
    ce = cond * ctx + (1.0 - cond) * mean_ctx

    mu2, inv2 = _gn_stats(ce, GC_ref, GCT_ref, T * (D // _GROUPS), 1e-6)
    ckn = (ce - mu2) * inv2 * cng_ref[...] + cnb_ref[...]
    k = jnp.dot(ckn, Wk_ref[...], preferred_element_type=F32) + bk_ref[...]

    ce_bf = ce.astype(BF16)
    v = bv_ref[...]
    for tap in range(9):
        sh = jnp.dot(SH16_ref[tap], ce_bf,
                     preferred_element_type=F32).astype(BF16)
        v = v + jnp.dot(sh, Wv_ref[tap], preferred_element_type=F32)

    logits = jax.lax.dot_general(q, k, (((1,), (1,)), ((), ())),
                                 preferred_element_type=F32)
    m = jnp.max(logits, axis=1, keepdims=True)
    e = jnp.exp(logits - m)
    attn = e / jnp.sum(e, axis=1, keepdims=True)
    attn_ref[0] = attn

    xa = jnp.dot(attn, v, preferred_element_type=F32)
    ac = jnp.dot(attn, ce, preferred_element_type=F32)

    mu3, inv3 = _gn_stats(xa, GL_ref, GLT_ref, _P * (C // _GROUPS), 1e-5)
    xn2 = (xa - mu3) * inv3
    ac_bf = ac.astype(BF16)
    sg = bsg_ref[...]
    sb = bsb_ref[...]
    for tap in range(9):
        sh = jnp.dot(SH9_ref[tap], ac_bf,
                     preferred_element_type=F32).astype(BF16)
        sg = sg + jnp.dot(sh, Wsg_ref[tap], preferred_element_type=F32)
        sb = sb + jnp.dot(sh, Wsb_ref[tap], preferred_element_type=F32)
    xlT = jnp.transpose(xn2 * sg + sb)

    gx = gx_ref[0]
    x1b = jnp.floor(bbox[:, 0:1] * 64.0)
    y1b = jnp.floor(bbox[:, 1:2] * 64.0)
    x2b = jnp.maximum(jnp.floor(bbox[:, 2:3] * 64.0), x1b + 1.0)
    y2b = jnp.maximum(jnp.floor(bbox[:, 3:4] * 64.0), y1b + 1.0)
    oh = y2b - y1b
    ow = x2b - x1b
    i3 = _iota((_ROI, _ROI, 4096), 0)
    j3 = _iota((_ROI, _ROI, 4096), 1)
    q3 = _iota((_ROI, _ROI, 4096), 2)
    y = jnp.floor(q3 * 0.015625)
    x = q3 - y * 64.0
    jy9 = (y - y1b) * 9.0
    jx9 = (x - x1b) * 9.0
    lo_y = i3 * oh
    lo_x = j3 * ow
    hit = ((jy9 >= lo_y) & (jy9 < lo_y + oh) & (y < y2b)
           & (jx9 >= lo_x) & (jx9 < lo_x + ow) & (x < x2b))
    Pm = jnp.where(hit, 1.0, 0.0).reshape(_P, 4096)
    out_ref[0] = gx + jnp.dot(xlT, Pm, preferred_element_type=F32)


def kernel(global_x, context, indicator, bbox, cond_flag,
           ln_g, ln_b, cn_g, cn_b, Wq, bq, Wk, bk, Wv, bv,
           Wsg, bsg, Wsb, bsb):
    B, C, H, W = global_x.shape
    T, D = context.shape[1], context.shape[2]
    HW = H * W

    gcorner = global_x[:, :, :8, :8].reshape(B, C, 64)
    gxf = global_x.reshape(B, C, HW)
    ind3 = indicator.reshape(B, 1, 2)
    bbox3 = bbox.reshape(B, 1, 4)
    cond3 = cond_flag.astype(F32).reshape(B, 1, 1)
    Wqm = Wq[:, :C, 0, 0].T
    Wqi = Wq[:, C:, 0, 0].T
    Wk2 = Wk[:, :, 0, 0].T
    Wv2 = Wv.transpose(2, 3, 1, 0).reshape(9, D, C).astype(BF16)
    Wsg2 = Wsg.transpose(2, 3, 1, 0).reshape(9, D, C).astype(BF16)
    Wsb2 = Wsb.transpose(2, 3, 1, 0).reshape(9, D, C).astype(BF16)
    row = lambda a: a.reshape(1, -1)

    consts = [jnp.asarray(_GL), jnp.asarray(_GL.T),
              jnp.asarray(_GC), jnp.asarray(_GC.T),
              jnp.asarray(_SH16, BF16), jnp.asarray(_SH9, BF16)]

    full = lambda s: pl.BlockSpec(s, lambda i, _n=None: (0,) * len(s))
    bat = lambda s: pl.BlockSpec((1,) + s, lambda i: (i,) + (0,) * len(s))

    cp = pltpu.CompilerParams(dimension_semantics=("parallel",),
                              vmem_limit_bytes=56 * 1024 * 1024)

    outf, attn = pl.pallas_call(
        _refine_kernel,
        grid=(B,),
        in_specs=[bat((C, 64)), bat((C, HW)), bat((T, D)), bat((1, 2)),
                  bat((1, 4)), bat((1, 1)),
                  full((C, C)), full((2, C)), full((1, C)),
                  full((D, C)), full((1, C)),
                  full((9, D, C)), full((1, C)),
                  full((9, D, C)), full((1, C)),
                  full((9, D, C)), full((1, C)),
                  full((1, C)), full((1, C)), full((1, D)), full((1, D)),
                  full((C, _GROUPS)), full((_GROUPS, C)),
                  full((D, _GROUPS)), full((_GROUPS, D)),
                  full((9, T, T)), full((9, _P, _P))],
        out_specs=[bat((C, HW)), bat((_P, T))],
        out_shape=[jax.ShapeDtypeStruct((B, C, HW), F32),
                   jax.ShapeDtypeStruct((B, _P, T), F32)],
        compiler_params=cp,
    )(gcorner, gxf, context, ind3, bbox3, cond3,
      Wqm, Wqi, row(bq), Wk2, row(bk),
      Wv2, row(bv), Wsg2, row(bsg), Wsb2, row(bsb),
      row(ln_g), row(ln_b), row(cn_g), row(cn_b), *consts)

    return outf.reshape(B, C, H, W), attn
